# baseline (speedup 1.0000x reference)
import functools

import jax
import jax.numpy as jnp
from jax import lax
from jax.experimental import pallas as pl
from jax.experimental.pallas import tpu as pltpu
from jax.experimental.pallas import tpu_sc as plsc

_B, _L, _V = 1024, 50, 1000
_N = _B * _L

_VP = 1024

_info = plsc.get_sparse_core_info()
_NC, _NS, _LANES = _info.num_cores, _info.num_subcores, _info.num_lanes
_NW = _NC * _NS
_RW = _N // _NW
_CH = 32
_NCH = _RW // _CH


def _lse_body(table_ref, out_ref):
    t = table_ref[...]
    m = jnp.max(t, axis=1)
    s = jnp.sum(jnp.exp(t - m[:, None]), axis=1)
    out_ref[...] = m + jnp.log(s)


def _row_lse(table):
    return pl.pallas_call(
        _lse_body,
        out_shape=jax.ShapeDtypeStruct((_V,), jnp.float32),
    )(table)


def _flat_body(table_hbm, tflat_hbm, stage_v, sem):
    t = lax.axis_index("s") * _NC + lax.axis_index("c")
    start = 31 * t + jnp.minimum(t, 8)
    count = jnp.where(t < 8, 32, 31)

    def fire_in(j, carry):
        @pl.when(j < count)
        def _():
            pltpu.async_copy(table_hbm.at[start + j], stage_v.at[j], sem)
        return carry

    def drain_in(j, carry):
        @pl.when(j < count)
        def _():
            pltpu.make_async_copy(table_hbm.at[0], stage_v.at[0], sem).wait()
        return carry

    def fire_out(j, carry):
        @pl.when(j < count)
        def _():
            pltpu.async_copy(stage_v.at[j],
                             tflat_hbm.at[pl.ds((start + j) * _V, _V)], sem)
        return carry

    def drain_out(j, carry):
        @pl.when(j < count)
        def _():
            pltpu.make_async_copy(stage_v.at[0],
                                  tflat_hbm.at[pl.ds(0, _V)], sem).wait()
        return carry

    lax.fori_loop(0, 32, fire_in, 0)
    lax.fori_loop(0, 32, drain_in, 0)
    lax.fori_loop(0, 32, fire_out, 0)
    lax.fori_loop(0, 32, drain_out, 0)


def _flatten_table(table):
    mesh = plsc.VectorSubcoreMesh(core_axis_name="c", subcore_axis_name="s")
    fn = pl.kernel(
        _flat_body,
        out_type=jax.ShapeDtypeStruct((_V * _V,), jnp.float32),
        mesh=mesh,
        compiler_params=pltpu.CompilerParams(use_tc_tiling_on_sc=False),
        scratch_types=[
            pltpu.VMEM((32, _V), jnp.float32),
            pltpu.SemaphoreType.DMA,
        ],
    )
    return fn(table)


def _sc_body(idx_hbm, tgt_hbm, table_hbm, tflat_hbm, lse_hbm,
             out_hbm, part_hbm,
             idx_v, rows0, rows1, tgt_c, comb_c, lse_c, tgtv_c, part_v,
             gsem0, gsem1, osem0, osem1, lsem):
    wid = lax.axis_index("s") * _NC + lax.axis_index("c")
    base = wid * _RW

    pltpu.sync_copy(idx_hbm.at[wid], idx_v)
    part_v[...] = jnp.zeros((_LANES,), jnp.float32)

    def fire_gather(c, buf, sem):
        pltpu.async_copy(table_hbm.at[idx_v.at[c]], buf, sem)

    def wait_gather(buf, sem):
        pltpu.make_async_copy(table_hbm.at[idx_v.at[0]], buf, sem).wait()

    def fire_out(c, buf, sem):
        pltpu.async_copy(buf, out_hbm.at[pl.ds(base + c * _CH, _CH)], sem)

    def wait_out(buf, sem):
        pltpu.make_async_copy(buf, out_hbm.at[pl.ds(base, _CH)], sem).wait()

    def loss_chunk(c):
        pltpu.sync_copy(tgt_hbm.at[wid, c], tgt_c)
        for j in range(_CH // _LANES):
            sl = pl.ds(j * _LANES, _LANES)
            comb_c[sl] = idx_v[c, sl] * _V + tgt_c[sl]
        pltpu.async_copy(lse_hbm.at[idx_v.at[c]], lse_c, lsem).wait()
        pltpu.async_copy(tflat_hbm.at[comb_c], tgtv_c, lsem).wait()
        acc = part_v[...]
        for j in range(_CH // _LANES):
            sl = pl.ds(j * _LANES, _LANES)
            acc = acc + (lse_c[sl] - tgtv_c[sl])
        part_v[...] = acc

    fire_gather(0, rows0, gsem0)
    fire_gather(1, rows1, gsem1)

    def pair(i, carry):
        c0 = 2 * i
        wait_gather(rows0, gsem0)
        fire_out(c0, rows0, osem0)
        loss_chunk(c0)
        wait_gather(rows1, gsem1)
        fire_out(c0 + 1, rows1, osem1)
        loss_chunk(c0 + 1)
        wait_out(rows0, osem0)
        wait_out(rows1, osem1)

        @pl.when(i < _NCH // 2 - 1)
        def _():
            fire_gather(c0 + 2, rows0, gsem0)
            fire_gather(c0 + 3, rows1, gsem1)

        return carry

    lax.fori_loop(0, _NCH // 2, pair, 0)
    pltpu.sync_copy(part_v, part_hbm.at[wid])


@functools.partial(jax.jit, static_argnums=())
def _sc_call(idx3, tgt3, table, tflat, lse_row):
    mesh = plsc.VectorSubcoreMesh(core_axis_name="c", subcore_axis_name="s")
    fn = pl.kernel(
        _sc_body,
        out_type=[
            jax.ShapeDtypeStruct((_N, _VP), jnp.float32),
            jax.ShapeDtypeStruct((_NW, _LANES), jnp.float32),
        ],
        mesh=mesh,
        compiler_params=pltpu.CompilerParams(use_tc_tiling_on_sc=False),
        scratch_types=[
            pltpu.VMEM((_NCH, _CH), jnp.int32),
            pltpu.VMEM((_CH, _VP), jnp.float32),
            pltpu.VMEM((_CH, _VP), jnp.float32),
            pltpu.VMEM((_CH,), jnp.int32),
            pltpu.VMEM((_CH,), jnp.int32),
            pltpu.VMEM((_CH,), jnp.float32),
            pltpu.VMEM((_CH,), jnp.float32),
            pltpu.VMEM((_LANES,), jnp.float32),
            pltpu.SemaphoreType.DMA,
            pltpu.SemaphoreType.DMA,
            pltpu.SemaphoreType.DMA,
            pltpu.SemaphoreType.DMA,
            pltpu.SemaphoreType.DMA,
        ],
    )
    return fn(idx3, tgt3, table, tflat, lse_row)


def kernel(input_b_l, target_b_1, embedding_table):
    idx3 = input_b_l.astype(jnp.int32).reshape(_NW, _NCH, _CH)
    tgt3 = target_b_1.astype(jnp.int32).reshape(_NW, _NCH, _CH)
    tflat = _flatten_table(embedding_table)
    lse_row = _row_lse(embedding_table)
    table_pad = jnp.pad(embedding_table, ((0, 0), (0, _VP - _V)))
    logits_pad, parts = _sc_call(idx3, tgt3, table_pad, tflat, lse_row)
    loss = jnp.sum(parts) / _N
    return logits_pad[:, :_V], loss


# speedup vs baseline: 1.5522x; 1.5522x over previous
import functools

import jax
import jax.numpy as jnp
from jax import lax
from jax.experimental import pallas as pl
from jax.experimental.pallas import tpu as pltpu
from jax.experimental.pallas import tpu_sc as plsc

_B, _L, _V = 1024, 50, 1000
_N = _B * _L
_VP = 1024

_info = plsc.get_sparse_core_info()
_NC, _NS, _LANES = _info.num_cores, _info.num_subcores, _info.num_lanes
_NW = _NC * _NS
_RW = _N // _NW
_GCH = 32
_GNCH = _RW // _GCH
_CH = 64
_NCH = _RW // _CH


def _lse_body(table_ref, out_ref):
    t = table_ref[...]
    m = jnp.max(t, axis=1)
    s = jnp.sum(jnp.exp(t - m[:, None]), axis=1)
    out_ref[...] = m + jnp.log(s)


def _row_lse(table):
    return pl.pallas_call(
        _lse_body,
        out_shape=jax.ShapeDtypeStruct((_V,), jnp.float32),
    )(table)


def _flat_body(table_hbm, tflat_hbm, stage_v, sem):
    t = lax.axis_index("s") * _NC + lax.axis_index("c")
    start = 31 * t + jnp.minimum(t, 8)
    count = jnp.where(t < 8, 32, 31)

    def fire_in(j, carry):
        @pl.when(j < count)
        def _():
            pltpu.async_copy(table_hbm.at[start + j], stage_v.at[j], sem)
        return carry

    def drain_in(j, carry):
        @pl.when(j < count)
        def _():
            pltpu.make_async_copy(table_hbm.at[0], stage_v.at[0], sem).wait()
        return carry

    def fire_out(j, carry):
        @pl.when(j < count)
        def _():
            pltpu.async_copy(stage_v.at[j],
                             tflat_hbm.at[pl.ds((start + j) * _V, _V)], sem)
        return carry

    def drain_out(j, carry):
        @pl.when(j < count)
        def _():
            pltpu.make_async_copy(stage_v.at[0],
                                  tflat_hbm.at[pl.ds(0, _V)], sem).wait()
        return carry

    lax.fori_loop(0, 32, fire_in, 0)
    lax.fori_loop(0, 32, drain_in, 0)
    lax.fori_loop(0, 32, fire_out, 0)
    lax.fori_loop(0, 32, drain_out, 0)


def _flatten_table(table):
    mesh = plsc.VectorSubcoreMesh(core_axis_name="c", subcore_axis_name="s")
    fn = pl.kernel(
        _flat_body,
        out_type=jax.ShapeDtypeStruct((_V * _V,), jnp.float32),
        mesh=mesh,
        compiler_params=pltpu.CompilerParams(use_tc_tiling_on_sc=False),
        scratch_types=[
            pltpu.VMEM((32, _V), jnp.float32),
            pltpu.SemaphoreType.DMA,
        ],
    )
    return fn(table)


def _gather_body(idx_hbm, table_hbm, out_hbm, idx_v, rows0, rows1,
                 gsem0, gsem1, osem0, osem1):
    wid = lax.axis_index("s") * _NC + lax.axis_index("c")
    base = wid * _RW
    pltpu.sync_copy(idx_hbm.at[wid], idx_v)

    def fire_gather(c, buf, sem):
        for j in range(_GCH // _LANES):
            iv = idx_v[pl.ds(c * _GCH + j * _LANES, _LANES)]
            pltpu.async_copy(table_hbm.at[iv],
                             buf.at[pl.ds(j * _LANES, _LANES)], sem)

    def wait_gather(buf, sem):
        for j in range(_GCH // _LANES):
            pltpu.make_async_copy(table_hbm.at[idx_v[pl.ds(0, _LANES)]],
                                  buf.at[pl.ds(0, _LANES)], sem).wait()

    def fire_out(c, buf, sem):
        pltpu.async_copy(buf, out_hbm.at[pl.ds(base + c * _GCH, _GCH)], sem)

    def wait_out(buf, sem):
        pltpu.make_async_copy(buf, out_hbm.at[pl.ds(base, _GCH)], sem).wait()

    fire_gather(0, rows0, gsem0)
    fire_gather(1, rows1, gsem1)

    def pair(i, carry):
        c0 = 2 * i
        wait_gather(rows0, gsem0)
        fire_out(c0, rows0, osem0)
        wait_gather(rows1, gsem1)
        fire_out(c0 + 1, rows1, osem1)
        wait_out(rows0, osem0)
        wait_out(rows1, osem1)

        @pl.when(i < _GNCH // 2 - 1)
        def _():
            fire_gather(c0 + 2, rows0, gsem0)
            fire_gather(c0 + 3, rows1, gsem1)

        return carry

    lax.fori_loop(0, _GNCH // 2, pair, 0)


@jax.jit
def _gather_call(idx2, table_pad):
    mesh = plsc.VectorSubcoreMesh(core_axis_name="c", subcore_axis_name="s")
    fn = pl.kernel(
        _gather_body,
        out_type=jax.ShapeDtypeStruct((_N, _VP), jnp.float32),
        mesh=mesh,
        scratch_types=[
            pltpu.VMEM((_RW,), jnp.int32),
            pltpu.VMEM((_GCH, _VP), jnp.float32),
            pltpu.VMEM((_GCH, _VP), jnp.float32),
            pltpu.SemaphoreType.DMA,
            pltpu.SemaphoreType.DMA,
            pltpu.SemaphoreType.DMA,
            pltpu.SemaphoreType.DMA,
        ],
    )
    return fn(idx2, table_pad)


def _loss_body(idx_hbm, tgt_hbm, tflat_hbm, lse_hbm, part_hbm,
               idx_v, tgt_c, comb_c, lse_c, tgtv_c, part_v, lsem):
    wid = lax.axis_index("s") * _NC + lax.axis_index("c")
    pltpu.sync_copy(idx_hbm.at[wid], idx_v)
    part_v[...] = jnp.zeros((_LANES,), jnp.float32)

    def chunk(c, carry):
        pltpu.sync_copy(tgt_hbm.at[wid, c], tgt_c)
        for j in range(_CH // _LANES):
            sl = pl.ds(j * _LANES, _LANES)
            comb_c[sl] = idx_v[c, sl] * _V + tgt_c[sl]
        pltpu.async_copy(lse_hbm.at[idx_v.at[c]], lse_c, lsem).wait()
        pltpu.async_copy(tflat_hbm.at[comb_c], tgtv_c, lsem).wait()
        acc = part_v[...]
        for j in range(_CH // _LANES):
            sl = pl.ds(j * _LANES, _LANES)
            acc = acc + (lse_c[sl] - tgtv_c[sl])
        part_v[...] = acc
        return carry

    lax.fori_loop(0, _NCH, chunk, 0)
    pltpu.sync_copy(part_v, part_hbm.at[wid])


@jax.jit
def _loss_call(idx3, tgt3, tflat, lse_row):
    mesh = plsc.VectorSubcoreMesh(core_axis_name="c", subcore_axis_name="s")
    fn = pl.kernel(
        _loss_body,
        out_type=jax.ShapeDtypeStruct((_NW, _LANES), jnp.float32),
        mesh=mesh,
        compiler_params=pltpu.CompilerParams(use_tc_tiling_on_sc=False),
        scratch_types=[
            pltpu.VMEM((_NCH, _CH), jnp.int32),
            pltpu.VMEM((_CH,), jnp.int32),
            pltpu.VMEM((_CH,), jnp.int32),
            pltpu.VMEM((_CH,), jnp.float32),
            pltpu.VMEM((_CH,), jnp.float32),
            pltpu.VMEM((_LANES,), jnp.float32),
            pltpu.SemaphoreType.DMA,
        ],
    )
    return fn(idx3, tgt3, tflat, lse_row)


def kernel(input_b_l, target_b_1, embedding_table):
    idx2 = input_b_l.astype(jnp.int32).reshape(_NW, _RW)
    idx3 = input_b_l.astype(jnp.int32).reshape(_NW, _NCH, _CH)
    tgt3 = target_b_1.astype(jnp.int32).reshape(_NW, _NCH, _CH)
    tflat = _flatten_table(embedding_table)
    lse_row = _row_lse(embedding_table)
    table_pad = jnp.pad(embedding_table, ((0, 0), (0, _VP - _V)))
    logits_pad = _gather_call(idx2, table_pad)
    parts = _loss_call(idx3, tgt3, tflat, lse_row)
    loss = jnp.sum(parts) / _N
    return logits_pad[:, :_V], loss
